# baseline (speedup 1.0000x reference)
import functools

import jax
import jax.numpy as jnp
from jax import lax
from jax.experimental import pallas as pl
from jax.experimental.pallas import tpu as pltpu
from jax.experimental.pallas import tpu_sc as plsc

USER_COUNT = 1000000
ITEM_COUNT = 1000000
DIM = 32
BATCH = 16384
PAD = 128

NUM_CORES = 2
NUM_SUBCORES = 16
NUM_WORKERS = NUM_CORES * NUM_SUBCORES
BPW = BATCH // NUM_WORKERS
L = 16
WAVE = 4
KC = 64
NWAVE = BPW // WAVE


def _g_body(u_hbm, p_hbm, n_hbm, ut_hbm, it_hbm,
            ou, oi, oj,
            iu, ip, inn, su, sp, sn, bu, bp, bn, semu, semp, semn):
    wid = lax.axis_index("s") * NUM_CORES + lax.axis_index("c")
    base = wid * BPW
    lane_iota = lax.iota(jnp.int32, L)

    streams = (
        (iu, ut_hbm, su, bu, ou, semu),
        (ip, it_hbm, sp, bp, oi, semp),
        (inn, it_hbm, sn, bn, oj, semn),
    )

    pltpu.sync_copy(u_hbm.at[pl.ds(base, BPW)], iu)
    pltpu.sync_copy(p_hbm.at[pl.ds(base, BPW)], ip)
    pltpu.sync_copy(n_hbm.at[pl.ds(base, BPW)], inn)

    def row_group(idxv, k):
        gb = pl.multiple_of((k >> 4) << 4, L)
        return idxv[pl.ds(gb, L)]

    def enqueue(idxv, table, stg, sem, w, half):
        for j in range(WAVE):
            k = w * WAVE + j
            grp = row_group(idxv, k)
            r = jnp.sum(jnp.where(lane_iota == (k & (L - 1)), grp, 0))
            tc = pl.multiple_of((r >> 7) << 7, 128)
            pltpu.async_copy(table.at[:, pl.ds(tc, 128)],
                             stg.at[half, j], sem)

    def extract(idxv, table, stg, obuf, out, sem, w, half):
        for j in range(WAVE):
            pltpu.make_async_copy(
                table.at[:, pl.ds(0, 128)], stg.at[half, j], sem).wait()
        for j in range(WAVE):
            k = w * WAVE + j
            grp = row_group(idxv, k)
            lane = jnp.full((L,), k & (L - 1), jnp.int32)
            lvec = lax.gather(
                grp, lane[:, None],
                dimension_numbers=lax.GatherDimensionNumbers(
                    offset_dims=(), collapsed_slice_dims=(0,),
                    start_index_map=(0,)),
                slice_sizes=(1,),
                mode=lax.GatherScatterMode.PROMISE_IN_BOUNDS) & 127
            kk = k & (KC - 1)
            lo = plsc.load_gather(stg.at[half, j], [lane_iota, lvec])
            hi = plsc.load_gather(stg.at[half, j], [lane_iota + L, lvec])
            obuf[kk, pl.ds(0, L)] = lo
            obuf[kk, pl.ds(L, L)] = hi

        @pl.when(lax.rem(w, KC // WAVE) == KC // WAVE - 1)
        def _():
            cb = (w // (KC // WAVE)) * KC
            pltpu.sync_copy(obuf, out.at[pl.ds(base + cb, KC)])

    for idxv, table, stg, _, _, sem in streams:
        enqueue(idxv, table, stg, sem, 0, 0)

    def pair(t, carry):
        w = t * 2
        for idxv, table, stg, obuf, out, sem in streams:
            enqueue(idxv, table, stg, sem, w + 1, 1)
            extract(idxv, table, stg, obuf, out, sem, w, 0)

            @pl.when(w + 2 < NWAVE)
            def _():
                enqueue(idxv, table, stg, sem, w + 2, 0)

            extract(idxv, table, stg, obuf, out, sem, w + 1, 1)
        return carry

    lax.fori_loop(0, NWAVE // 2, pair, 0)


@jax.jit
def kernel(u, p, n, user_emb, item_emb):
    u = jnp.asarray(u, jnp.int32)
    p = jnp.asarray(p, jnp.int32)
    n = jnp.asarray(n, jnp.int32)
    ut = user_emb.T
    it = item_emb.T
    mesh = plsc.VectorSubcoreMesh(
        core_axis_name="c", subcore_axis_name="s",
        num_cores=NUM_CORES, num_subcores=NUM_SUBCORES)
    out = jax.ShapeDtypeStruct((BATCH, PAD), jnp.float32)
    idx_t = pltpu.VMEM((BPW,), jnp.int32)
    stg_t = pltpu.VMEM((2, WAVE, DIM, 128), jnp.float32)
    obuf_t = pltpu.VMEM((KC, PAD), jnp.float32)
    run = pl.kernel(
        _g_body,
        out_type=(out, out, out),
        mesh=mesh,
        scratch_types=[
            idx_t, idx_t, idx_t,
            stg_t, stg_t, stg_t,
            obuf_t, obuf_t, obuf_t,
            pltpu.SemaphoreType.DMA,
            pltpu.SemaphoreType.DMA,
            pltpu.SemaphoreType.DMA,
        ],
        compiler_params=pltpu.CompilerParams(needs_layout_passes=False),
    )
    ou, oi, oj = run(u, p, n, ut, it)
    return (ou[:, :DIM], oi[:, :DIM], oj[:, :DIM])


# speedup vs baseline: 1.0206x; 1.0206x over previous
import functools

import jax
import jax.numpy as jnp
from jax import lax
from jax.experimental import pallas as pl
from jax.experimental.pallas import tpu as pltpu
from jax.experimental.pallas import tpu_sc as plsc

USER_COUNT = 1000000
ITEM_COUNT = 1000000
DIM = 32
BATCH = 16384

NUM_CORES = 2
NUM_SUBCORES = 16
NUM_WORKERS = NUM_CORES * NUM_SUBCORES
BPW = BATCH // NUM_WORKERS
L = 16
WAVE = 16
KC = 128
NWAVE = BPW // WAVE


def _g_body(u_hbm, p_hbm, n_hbm, ut_hbm, it_hbm,
            ou, oi, oj, idxv, stg, obuf, sem):
    wid = lax.axis_index("s") * NUM_CORES + lax.axis_index("c")
    base = wid * BPW
    lane_iota = lax.iota(jnp.int32, L)

    def lookup(idx_hbm, table, out):
        pltpu.sync_copy(idx_hbm.at[pl.ds(base, BPW)], idxv)

        def enqueue(w):
            for j in range(WAVE):
                k = w * WAVE + j
                grp = idxv[pl.ds(pl.multiple_of((k >> 4) << 4, L), L)]
                r = jnp.sum(jnp.where(lane_iota == (k & (L - 1)), grp, 0))
                tc = pl.multiple_of((r >> 7) << 7, 128)
                pltpu.async_copy(table.at[:, pl.ds(tc, 128)],
                                 stg.at[j], sem)

        enqueue(0)

        def wave(w, carry):
            pltpu.make_async_copy(
                table.at[:, pl.ds(0, 128 * WAVE)], stg, sem).wait()
            lvec = idxv[pl.ds(pl.multiple_of(w * WAVE, L), L)] & 127
            k0 = lax.rem(w, KC // WAVE) * WAVE
            for c in range(DIM):
                cvec = jnp.full((L,), c, jnp.int32)
                v = plsc.load_gather(stg, [lane_iota, cvec, lvec])
                obuf[c, pl.ds(k0, L)] = v

            @pl.when(lax.rem(w, KC // WAVE) == KC // WAVE - 1)
            def _():
                cb = (w // (KC // WAVE)) * KC
                pltpu.sync_copy(
                    obuf, out.at[:, pl.ds(pl.multiple_of(base + cb, 128),
                                          KC)])

            @pl.when(w < NWAVE - 1)
            def _():
                enqueue(w + 1)

            return carry

        lax.fori_loop(0, NWAVE, wave, 0)

    lookup(u_hbm, ut_hbm, ou)
    lookup(p_hbm, it_hbm, oi)
    lookup(n_hbm, it_hbm, oj)


@jax.jit
def kernel(u, p, n, user_emb, item_emb):
    u = jnp.asarray(u, jnp.int32)
    p = jnp.asarray(p, jnp.int32)
    n = jnp.asarray(n, jnp.int32)
    ut = user_emb.T
    it = item_emb.T
    mesh = plsc.VectorSubcoreMesh(
        core_axis_name="c", subcore_axis_name="s",
        num_cores=NUM_CORES, num_subcores=NUM_SUBCORES)
    out = jax.ShapeDtypeStruct((DIM, BATCH), jnp.float32)
    run = pl.kernel(
        _g_body,
        out_type=(out, out, out),
        mesh=mesh,
        scratch_types=[
            pltpu.VMEM((BPW,), jnp.int32),
            pltpu.VMEM((WAVE, DIM, 128), jnp.float32),
            pltpu.VMEM((DIM, KC), jnp.float32),
            pltpu.SemaphoreType.DMA,
        ],
        compiler_params=pltpu.CompilerParams(needs_layout_passes=False),
    )
    ou, oi, oj = run(u, p, n, ut, it)
    return (ou.T, oi.T, oj.T)


# speedup vs baseline: 1.0624x; 1.0410x over previous
import functools

import jax
import jax.numpy as jnp
from jax import lax
from jax.experimental import pallas as pl
from jax.experimental.pallas import tpu as pltpu
from jax.experimental.pallas import tpu_sc as plsc

USER_COUNT = 1000000
ITEM_COUNT = 1000000
DIM = 32
BATCH = 16384
PAD = 128

NUM_CORES = 2
NUM_SUBCORES = 16
NUM_WORKERS = NUM_CORES * NUM_SUBCORES
BPW = BATCH // NUM_WORKERS
L = 16
WAVE = 8
KC = 64
NWAVE = BPW // WAVE


def _g_body(u_hbm, p_hbm, n_hbm, ut_hbm, it_hbm,
            ou, oi, oj,
            iu, ip, inn, su, sp, sn, bu, bp, bn, semu, semp, semn):
    wid = lax.axis_index("s") * NUM_CORES + lax.axis_index("c")
    base = wid * BPW
    lane_iota = lax.iota(jnp.int32, L)

    ut4 = ut_hbm.reshape(4, 8, USER_COUNT)
    it4 = it_hbm.reshape(4, 8, ITEM_COUNT)
    streams = (
        (iu, ut4, su, bu, ou, semu),
        (ip, it4, sp, bp, oi, semp),
        (inn, it4, sn, bn, oj, semn),
    )

    pltpu.sync_copy(u_hbm.at[pl.ds(base, BPW)], iu)
    pltpu.sync_copy(p_hbm.at[pl.ds(base, BPW)], ip)
    pltpu.sync_copy(n_hbm.at[pl.ds(base, BPW)], inn)

    def row_scalar(idxv, k):
        gb = pl.multiple_of((k >> 4) << 4, L)
        grp = idxv[pl.ds(gb, L)]
        return jnp.sum(jnp.where(lane_iota == (k & (L - 1)), grp, 0))

    def enqueue(idxv, table, stg, sem, w):
        for j in range(WAVE):
            r = row_scalar(idxv, w * WAVE + j)
            tc = pl.multiple_of((r >> 7) << 7, 128)
            for b in range(4):
                pltpu.async_copy(table.at[b, :, pl.ds(tc, 128)],
                                 stg.at[j].at[pl.ds(b * 8, 8)], sem)

    for idxv, table, stg, _, _, sem in streams:
        enqueue(idxv, table, stg, sem, 0)

    def wave(w, carry):
        for idxv, table, stg, obuf, out, sem in streams:
            for j in range(WAVE):
                pltpu.make_async_copy(
                    table.at[0, :, pl.ds(0, 128)], stg.at[j], sem).wait()
            for j in range(WAVE):
                r = row_scalar(idxv, w * WAVE + j)
                lvec = jnp.broadcast_to(r & 127, (L,))
                k = (w * WAVE + j) & (KC - 1)
                lo = plsc.load_gather(stg.at[j], [lane_iota, lvec])
                hi = plsc.load_gather(stg.at[j], [lane_iota + L, lvec])
                obuf[k, pl.ds(0, L)] = lo
                obuf[k, pl.ds(L, L)] = hi

            @pl.when(w < NWAVE - 1)
            def _():
                enqueue(idxv, table, stg, sem, w + 1)

            @pl.when(lax.rem(w, KC // WAVE) == KC // WAVE - 1)
            def _():
                cb = (w // (KC // WAVE)) * KC
                pltpu.sync_copy(obuf, out.at[pl.ds(base + cb, KC)])
        return carry

    lax.fori_loop(0, NWAVE, wave, 0)


@jax.jit
def kernel(u, p, n, user_emb, item_emb):
    u = jnp.asarray(u, jnp.int32)
    p = jnp.asarray(p, jnp.int32)
    n = jnp.asarray(n, jnp.int32)
    ut = user_emb.T
    it = item_emb.T
    mesh = plsc.VectorSubcoreMesh(
        core_axis_name="c", subcore_axis_name="s",
        num_cores=NUM_CORES, num_subcores=NUM_SUBCORES)
    out = jax.ShapeDtypeStruct((BATCH, PAD), jnp.float32)
    idx_t = pltpu.VMEM((BPW,), jnp.int32)
    stg_t = pltpu.VMEM((WAVE, DIM, 128), jnp.float32)
    obuf_t = pltpu.VMEM((KC, PAD), jnp.float32)
    run = pl.kernel(
        _g_body,
        out_type=(out, out, out),
        mesh=mesh,
        scratch_types=[
            idx_t, idx_t, idx_t,
            stg_t, stg_t, stg_t,
            obuf_t, obuf_t, obuf_t,
            pltpu.SemaphoreType.DMA,
            pltpu.SemaphoreType.DMA,
            pltpu.SemaphoreType.DMA,
        ],
        compiler_params=pltpu.CompilerParams(needs_layout_passes=False),
    )
    ou, oi, oj = run(u, p, n, ut, it)
    return (ou[:, :DIM], oi[:, :DIM], oj[:, :DIM])


# speedup vs baseline: 1.0871x; 1.0232x over previous
import functools

import jax
import jax.numpy as jnp
from jax import lax
from jax.experimental import pallas as pl
from jax.experimental.pallas import tpu as pltpu
from jax.experimental.pallas import tpu_sc as plsc

USER_COUNT = 1000000
ITEM_COUNT = 1000000
DIM = 32
BATCH = 16384
PAD = 128

NUM_CORES = 2
NUM_SUBCORES = 16
NUM_WORKERS = NUM_CORES * NUM_SUBCORES
BPW = BATCH // NUM_WORKERS
L = 16
WAVE = 8
KC = 128
NWAVE = BPW // WAVE


def _g_body(u_hbm, p_hbm, n_hbm, ut_hbm, it_hbm,
            ou, oi, oj,
            iu, ip, inn, su, sp, sn, bu, bp, bn, semu, semp, semn):
    wid = lax.axis_index("s") * NUM_CORES + lax.axis_index("c")
    base = wid * BPW
    lane_iota = lax.iota(jnp.int32, L)

    ut4 = ut_hbm.reshape(4, 8, USER_COUNT)
    it4 = it_hbm.reshape(4, 8, ITEM_COUNT)
    streams = (
        (iu, ut4, su, bu, ou, semu),
        (ip, it4, sp, bp, oi, semp),
        (inn, it4, sn, bn, oj, semn),
    )

    pltpu.sync_copy(u_hbm.at[pl.ds(base, BPW)], iu)
    pltpu.sync_copy(p_hbm.at[pl.ds(base, BPW)], ip)
    pltpu.sync_copy(n_hbm.at[pl.ds(base, BPW)], inn)

    def row_scalar(idxv, k):
        gb = pl.multiple_of((k >> 4) << 4, L)
        grp = idxv[pl.ds(gb, L)]
        return jnp.sum(jnp.where(lane_iota == (k & (L - 1)), grp, 0))

    def enqueue(idxv, table, stg, sem, w):
        for j in range(WAVE):
            r = row_scalar(idxv, w * WAVE + j)
            tc = pl.multiple_of((r >> 7) << 7, 128)
            for b in range(4):
                pltpu.async_copy(table.at[b, :, pl.ds(tc, 128)],
                                 stg.at[j].at[pl.ds(b * 8, 8)], sem)

    for idxv, table, stg, _, _, sem in streams:
        enqueue(idxv, table, stg, sem, 0)

    def wave(w, carry):
        for idxv, table, stg, obuf, out, sem in streams:
            for j in range(WAVE):
                pltpu.make_async_copy(
                    table.at[0, :, pl.ds(0, 128)], stg.at[j], sem).wait()
            for j in range(WAVE):
                r = row_scalar(idxv, w * WAVE + j)
                lvec = jnp.broadcast_to(r & 127, (L,))
                k = (w * WAVE + j) & (KC - 1)
                kvec = jnp.full((L,), k, jnp.int32)
                lo = plsc.load_gather(stg.at[j], [lane_iota, lvec])
                hi = plsc.load_gather(stg.at[j], [lane_iota + L, lvec])
                plsc.store_scatter(obuf, [lane_iota, kvec], lo)
                plsc.store_scatter(obuf, [lane_iota + L, kvec], hi)

            @pl.when(w < NWAVE - 1)
            def _():
                enqueue(idxv, table, stg, sem, w + 1)

            @pl.when(lax.rem(w, KC // WAVE) == KC // WAVE - 1)
            def _():
                cb = (w // (KC // WAVE)) * KC
                pltpu.sync_copy(
                    obuf,
                    out.at[:, pl.ds(pl.multiple_of(base + cb, 128), KC)])
        return carry

    lax.fori_loop(0, NWAVE, wave, 0)


@jax.jit
def kernel(u, p, n, user_emb, item_emb):
    u = jnp.asarray(u, jnp.int32)
    p = jnp.asarray(p, jnp.int32)
    n = jnp.asarray(n, jnp.int32)
    ut = user_emb.T
    it = item_emb.T
    mesh = plsc.VectorSubcoreMesh(
        core_axis_name="c", subcore_axis_name="s",
        num_cores=NUM_CORES, num_subcores=NUM_SUBCORES)
    out = jax.ShapeDtypeStruct((DIM, BATCH), jnp.float32)
    idx_t = pltpu.VMEM((BPW,), jnp.int32)
    stg_t = pltpu.VMEM((WAVE, DIM, 128), jnp.float32)
    obuf_t = pltpu.VMEM((DIM, KC), jnp.float32)
    run = pl.kernel(
        _g_body,
        out_type=(out, out, out),
        mesh=mesh,
        scratch_types=[
            idx_t, idx_t, idx_t,
            stg_t, stg_t, stg_t,
            obuf_t, obuf_t, obuf_t,
            pltpu.SemaphoreType.DMA,
            pltpu.SemaphoreType.DMA,
            pltpu.SemaphoreType.DMA,
        ],
        compiler_params=pltpu.CompilerParams(needs_layout_passes=False),
    )
    ou, oi, oj = run(u, p, n, ut, it)
    return (ou.T, oi.T, oj.T)
